# baseline (speedup 1.0000x reference)
import functools

import jax
import jax.numpy as jnp
from jax import lax
from jax.experimental import pallas as pl
from jax.experimental.pallas import tpu as pltpu
from jax.experimental.pallas import tpu_sc as plsc

B = 16384
EMB = 32
OUT_D = EMB + 1
VOCAB = 1000000
NC, NS, L = 2, 16, 16
NW = NC * NS
BW = B // NW
CHUNK = 128
NCH = BW // CHUNK

_mesh = plsc.VectorSubcoreMesh(core_axis_name="c", subcore_axis_name="s")


def _transpose_cm(cm_ref, rbuf_ref):
  lanes = lax.iota(jnp.int32, L)
  zeros = jnp.zeros((L,), jnp.int32)

  def group(g, _):
    lv = g * L + lanes
    for c in range(EMB):
      vals = plsc.load_gather(cm_ref, [zeros + c, lv])
      plsc.store_scatter(rbuf_ref, [lv, zeros + c], vals)
    return 0

  lax.fori_loop(0, CHUNK // L, group, 0, unroll=False)


def _gather_body(uids, iids, utT, itT, uemb, iemb,
                 uidx, iidx, ucm, icm, urbuf, irbuf, usem, isem):
  wid = lax.axis_index("s") * NC + lax.axis_index("c")
  base = wid * BW

  pltpu.sync_copy(uids.at[pl.ds(base, BW)], uidx)
  pltpu.sync_copy(iids.at[pl.ds(base, BW)], iidx)

  def fire(tT, idx, cm, sem, j):
    return [
        pltpu.async_copy(tT.at[c].at[idx.at[pl.ds(j * CHUNK, CHUNK)]],
                         cm.at[c], sem)
        for c in range(EMB)
    ]

  for j in range(NCH):
    ucopies = fire(utT, uidx, ucm, usem, j)
    icopies = fire(itT, iidx, icm, isem, j)
    for cp in ucopies:
      cp.wait()
    _transpose_cm(ucm, urbuf)
    pltpu.sync_copy(urbuf, uemb.at[pl.ds(base + j * CHUNK, CHUNK)])
    for cp in icopies:
      cp.wait()
    _transpose_cm(icm, irbuf)
    pltpu.sync_copy(irbuf, iemb.at[pl.ds(base + j * CHUNK, CHUNK)])


_gather_call = functools.partial(
    pl.kernel,
    out_type=[
        jax.ShapeDtypeStruct((B, EMB), jnp.float32),
        jax.ShapeDtypeStruct((B, EMB), jnp.float32),
    ],
    mesh=_mesh,
    scratch_types=[
        pltpu.VMEM((BW,), jnp.int32),
        pltpu.VMEM((BW,), jnp.int32),
        pltpu.VMEM((EMB, CHUNK), jnp.float32),
        pltpu.VMEM((EMB, CHUNK), jnp.float32),
        pltpu.VMEM((CHUNK, EMB), jnp.float32),
        pltpu.VMEM((CHUNK, EMB), jnp.float32),
        pltpu.SemaphoreType.DMA,
        pltpu.SemaphoreType.DMA,
    ],
    compiler_params=pltpu.CompilerParams(use_tc_tiling_on_sc=False,
                                         needs_layout_passes=False),
)(_gather_body)


def _assemble_chunk(idx_ref, rbuf, feat_ref, j):
  lanes = lax.iota(jnp.int32, L)
  zeros = jnp.zeros((L,), jnp.int32)

  def group(g, _):
    sv = j * CHUNK + g * L + lanes
    lid = g * L + lanes
    ids = plsc.load_gather(idx_ref, [sv])
    plsc.store_scatter(feat_ref, [zeros, sv], ids.astype(jnp.float32))
    for r in range(EMB):
      vals = plsc.load_gather(rbuf, [lid, zeros + r])
      plsc.store_scatter(feat_ref, [zeros + (1 + r), sv], vals)
    return 0

  lax.fori_loop(0, CHUNK // L, group, 0, unroll=False)


def _assemble_body(uids, iids, uemb, iemb, uoutT, ioutT,
                   uidx, iidx, ubuf, ibuf, ufeat, ifeat, usems, isems):
  wid = lax.axis_index("s") * NC + lax.axis_index("c")
  base = wid * BW

  pltpu.sync_copy(uids.at[pl.ds(base, BW)], uidx)
  pltpu.sync_copy(iids.at[pl.ds(base, BW)], iidx)

  def fire(src, buf, sems, j):
    return pltpu.async_copy(src.at[pl.ds(base + j * CHUNK, CHUNK)],
                            buf.at[j % 2], sems.at[j % 2])

  ucopies = [fire(uemb, ubuf, usems, j) for j in range(2)]
  icopies = [fire(iemb, ibuf, isems, j) for j in range(2)]

  for j in range(NCH):
    ucopies[j].wait()
    _assemble_chunk(uidx, ubuf.at[j % 2], ufeat, j)
    if j + 2 < NCH:
      ucopies.append(fire(uemb, ubuf, usems, j + 2))
  pltpu.sync_copy(ufeat, uoutT.at[:, pl.ds(base, BW)])

  for j in range(NCH):
    icopies[j].wait()
    _assemble_chunk(iidx, ibuf.at[j % 2], ifeat, j)
    if j + 2 < NCH:
      icopies.append(fire(iemb, ibuf, isems, j + 2))
  pltpu.sync_copy(ifeat, ioutT.at[:, pl.ds(base, BW)])


_assemble_call = functools.partial(
    pl.kernel,
    out_type=[
        jax.ShapeDtypeStruct((OUT_D, B), jnp.float32),
        jax.ShapeDtypeStruct((OUT_D, B), jnp.float32),
    ],
    mesh=_mesh,
    scratch_types=[
        pltpu.VMEM((BW,), jnp.int32),
        pltpu.VMEM((BW,), jnp.int32),
        pltpu.VMEM((2, CHUNK, EMB), jnp.float32),
        pltpu.VMEM((2, CHUNK, EMB), jnp.float32),
        pltpu.VMEM((OUT_D, BW), jnp.float32),
        pltpu.VMEM((OUT_D, BW), jnp.float32),
        pltpu.SemaphoreType.DMA((2,)),
        pltpu.SemaphoreType.DMA((2,)),
    ],
    compiler_params=pltpu.CompilerParams(needs_layout_passes=False),
)(_assemble_body)


@jax.jit
def kernel(user_id, item_id, user_table, item_table):
  uids = user_id.reshape(B).astype(jnp.int32)
  iids = item_id.reshape(B).astype(jnp.int32)
  uemb, iemb = _gather_call(uids, iids, user_table.T, item_table.T)
  uT, iT = _assemble_call(uids, iids, uemb, iemb)
  return uT.T, iT.T
